# baseline (speedup 1.0000x reference)
import jax
import jax.numpy as jnp
from jax.experimental import pallas as pl
from jax.experimental.pallas import tpu as pltpu

E = 32
CHUNK = 256000
CH2 = CHUNK // 128
NBUF = 8


def _swap(x, msk):
    return jnp.where(msk, pltpu.roll(x, 3 * E, axis=1),
                     pltpu.roll(x, E, axis=1))


def _body(in0_hbm, in1_hbm, out_hbm, x0, x1, ob, sem_in, sem_out):
    s = pl.program_id(0)
    S = pl.num_programs(0)

    def buf(ref, slot):
        return ref.at[pl.ds(slot * CHUNK, CHUNK)]

    def in_copies(step, slot):
        el = pl.ds(step * CHUNK, CHUNK)
        return (
            pltpu.make_async_copy(in0_hbm.at[el], buf(x0, slot),
                                  sem_in.at[slot, 0]),
            pltpu.make_async_copy(in1_hbm.at[el], buf(x1, slot),
                                  sem_in.at[slot, 1]),
        )

    def out_copy(step, slot):
        el = pl.ds(step * CHUNK, CHUNK)
        return pltpu.make_async_copy(buf(ob, slot), out_hbm.at[el],
                                     sem_out.at[slot])

    slot = jax.lax.rem(s, NBUF)

    @pl.when(s == 0)
    def _prologue():
        for k in range(NBUF):
            for c in in_copies(k, k):
                c.start()

    for c in in_copies(s, slot):
        c.wait()

    @pl.when(s >= NBUF)
    def _wait_prev_out():
        out_copy(s - NBUF, slot).wait()

    a = x0[pl.ds(slot * CHUNK, CHUNK)].reshape(CH2, 128)
    b = x1[pl.ds(slot * CHUNK, CHUNK)].reshape(CH2, 128)
    lane = jax.lax.broadcasted_iota(jnp.int32, (CH2, 128), 1)
    msk = (lane % (2 * E)) < E
    p = a * b
    q = _swap(a, msk) * b
    z = jnp.where(msk, p, q)
    w = jnp.where(msk, q, -p)
    r = z + _swap(w, msk)
    ob[pl.ds(slot * CHUNK, CHUNK)] = r.reshape(CHUNK)

    out_copy(s, slot).start()

    nxt = s + NBUF

    @pl.when(nxt < S)
    def _prefetch():
        for c in in_copies(nxt, jax.lax.rem(nxt, NBUF)):
            c.start()

    @pl.when(s == S - 1)
    def _epilogue():
        for k in range(NBUF):
            step = S - NBUF + k
            out_copy(step, jax.lax.rem(step, NBUF)).wait()


def kernel(in0, in1):
    n, d = in0.shape
    tot = n * d
    S = tot // CHUNK
    scratch = (
        [pltpu.VMEM((NBUF * CHUNK,), jnp.float32) for _ in range(3)]
        + [pltpu.SemaphoreType.DMA((NBUF, 2)),
           pltpu.SemaphoreType.DMA((NBUF,))]
    )
    out = pl.pallas_call(
        _body,
        grid=(S,),
        in_specs=[pl.BlockSpec(memory_space=pl.ANY)] * 2,
        out_specs=pl.BlockSpec(memory_space=pl.ANY),
        out_shape=jax.ShapeDtypeStruct((tot,), jnp.float32),
        scratch_shapes=scratch,
    )(in0.reshape(tot), in1.reshape(tot))
    return out.reshape(n, d)


# speedup vs baseline: 1.2651x; 1.2651x over previous
import jax
import jax.numpy as jnp
from jax.experimental import pallas as pl
from jax.experimental.pallas import tpu as pltpu

E = 32
R = 4000
NBUF = 4


def _body(in0_hbm, in1_hbm, out_hbm, x0, x1, ob, sem_in, sem_out):
    s = pl.program_id(0)
    S = pl.num_programs(0)

    def in_copies(step, slot):
        rows = pl.ds(step * R, R)
        return (
            pltpu.make_async_copy(in0_hbm.at[rows], x0.at[slot],
                                  sem_in.at[slot, 0]),
            pltpu.make_async_copy(in1_hbm.at[rows], x1.at[slot],
                                  sem_in.at[slot, 1]),
        )

    def start_in(step, slot):
        c0, c1 = in_copies(step, slot)
        even = jax.lax.rem(step, 2) == 0

        @pl.when(even)
        def _():
            c0.start(priority=0)
            c1.start(priority=1)

        c0b, c1b = in_copies(step, slot)

        @pl.when(jnp.logical_not(even))
        def _():
            c0b.start(priority=1)
            c1b.start(priority=0)

    def out_copy(step, slot):
        rows = pl.ds(step * R, R)
        return pltpu.make_async_copy(ob.at[slot], out_hbm.at[rows],
                                     sem_out.at[slot])

    slot = jax.lax.rem(s, NBUF)

    @pl.when(s == 0)
    def _prologue():
        for k in range(NBUF):
            start_in(k, k)

    for c in in_copies(s, slot):
        c.wait()

    @pl.when(s >= NBUF)
    def _wait_prev_out():
        out_copy(s - NBUF, slot).wait()

    a = x0[slot]
    b = x1[slot]
    ar = a[:, :E]
    ai = a[:, E:]
    br = b[:, :E]
    bi = b[:, E:]
    ob[slot] = jnp.concatenate([ar * br - ai * bi, ai * br + ar * bi],
                               axis=1)

    even_s = jax.lax.rem(s, 2) == 0

    @pl.when(even_s)
    def _start_out_even():
        out_copy(s, slot).start(priority=0)

    @pl.when(jnp.logical_not(even_s))
    def _start_out_odd():
        out_copy(s, slot).start(priority=1)

    nxt = s + NBUF

    @pl.when(nxt < S)
    def _prefetch():
        start_in(nxt, jax.lax.rem(nxt, NBUF))

    @pl.when(s == S - 1)
    def _epilogue():
        for k in range(NBUF):
            step = S - NBUF + k
            out_copy(step, jax.lax.rem(step, NBUF)).wait()


def kernel(in0, in1):
    n, d = in0.shape
    S = n // R
    scratch = (
        [pltpu.VMEM((NBUF, R, d), jnp.float32) for _ in range(3)]
        + [pltpu.SemaphoreType.DMA((NBUF, 2)),
           pltpu.SemaphoreType.DMA((NBUF,))]
    )
    return pl.pallas_call(
        _body,
        grid=(S,),
        in_specs=[pl.BlockSpec(memory_space=pl.ANY)] * 2,
        out_specs=pl.BlockSpec(memory_space=pl.ANY),
        out_shape=jax.ShapeDtypeStruct((n, d), jnp.float32),
        scratch_shapes=scratch,
    )(in0, in1)
